# Initial kernel scaffold; baseline (speedup 1.0000x reference)
import jax
import jax.numpy as jnp
from jax.experimental import pallas as pl


def kernel(edge_index, edge_values, item_emb):
    raise NotImplementedError("write your pallas kernel here")


# speedup vs baseline: 3.6654x; 3.6654x over previous
import jax
import jax.numpy as jnp
from jax import lax
from jax.experimental import pallas as pl
from jax.experimental.pallas import tpu as pltpu
from jax.experimental.pallas import tpu_sc as plsc

N = 10001
D = 128
E = 320000
NP = 10240
NC = 2
NS = 16
C = 80
SUP = 25
SUPE = SUP * C
EW1 = E // (NC * NS)
NSUP1 = EW1 // SUPE
EW2 = E // NS
NSUP2 = EW2 // SUPE
NPS = NP // NS
L = 16


def _sc_body(emb_p, row4a, col4a, row4b, col4b, vals, zeros,
             p_out, q_out, acc, rowb, colb, valb, gbuf, sem):
    c = lax.axis_index("c")
    s = lax.axis_index("s")
    w = c * NS + s
    rbase = s * NPS
    half = c * NP

    pltpu.sync_copy(zeros.at[pl.ds(rbase, NPS)], acc.at[pl.ds(rbase, NPS)])
    plsc.subcore_barrier()

    offv = jnp.broadcast_to(half, (L,)).astype(jnp.int32)

    def spmm_phase(row4, col4, widx, vbase, nsup, table, add_off):
        def sup(m, _):
            pltpu.sync_copy(row4.at[widx, m], rowb)
            pltpu.sync_copy(col4.at[widx, m], colb)
            pltpu.sync_copy(vals.at[pl.ds(vbase + m * SUPE, SUPE)],
                            valb.at[pl.ds(0, SUPE)])

            if add_off:
                def adj_row(k, _):
                    def adj_grp(j, _):
                        colb[k, pl.ds(j * L, L)] = (
                            colb[k, pl.ds(j * L, L)] + offv)
                        return 0
                    return lax.fori_loop(0, C // L, adj_grp, 0)
                lax.fori_loop(0, SUP, adj_row, 0)

            def chunk(k, _):
                pltpu.async_copy(table.at[colb.at[k]], gbuf, sem).wait()

                def edge(i, _):
                    vv = jnp.broadcast_to(valb[pl.ds(k * C + i, L)][0], (L,))
                    for j in range(D // L):
                        gbuf[i, pl.ds(j * L, L)] = (
                            gbuf[i, pl.ds(j * L, L)] * vv)
                    return 0

                lax.fori_loop(0, C, edge, 0)

                pltpu.sync_copy(gbuf, acc.at[rowb.at[k]], add=True)
                return 0

            lax.fori_loop(0, SUP, chunk, 0)
            return 0

        lax.fori_loop(0, nsup, sup, 0)
        plsc.subcore_barrier()

    spmm_phase(row4a, col4a, w, w * EW1, NSUP1, emb_p, False)

    pltpu.sync_copy(acc.at[pl.ds(rbase, NPS)],
                    p_out.at[pl.ds(half + rbase, NPS)])
    pltpu.sync_copy(zeros.at[pl.ds(rbase, NPS)], acc.at[pl.ds(rbase, NPS)])
    plsc.subcore_barrier()

    spmm_phase(row4b, col4b, s, s * EW2, NSUP2, p_out, True)

    pltpu.sync_copy(acc.at[pl.ds(rbase, NPS)],
                    q_out.at[pl.ds(half + rbase, NPS)])


def _tc_body(e0_ref, p0_ref, p1_ref, q0_ref, q1_ref,
             e1_ref, e2_ref, sum_ref):
    e1 = p0_ref[...] + p1_ref[...]
    e2 = q0_ref[...] + q1_ref[...]
    e1_ref[...] = e1
    e2_ref[...] = e2
    sum_ref[...] = e0_ref[...] + e1 + e2


@jax.jit
def _run(emb_p, row4a, col4a, row4b, col4b, vals, zeros):
    mesh = plsc.VectorSubcoreMesh(core_axis_name="c", subcore_axis_name="s")
    sc = pl.kernel(
        _sc_body,
        out_type=(
            jax.ShapeDtypeStruct((NC * NP, D), jnp.float32),
            jax.ShapeDtypeStruct((NC * NP, D), jnp.float32),
        ),
        mesh=mesh,
        scratch_types=[
            pltpu.VMEM_SHARED((NP, D), jnp.float32),
            pltpu.VMEM((SUP, C), jnp.int32),
            pltpu.VMEM((SUP, C), jnp.int32),
            pltpu.VMEM((SUPE + L,), jnp.float32),
            pltpu.VMEM((C, D), jnp.float32),
            pltpu.SemaphoreType.DMA,
        ],
    )
    p_out, q_out = sc(emb_p, row4a, col4a, row4b, col4b, vals, zeros)

    blk = 512
    grid = (NP // blk,)
    spec0 = pl.BlockSpec((blk, D), lambda i: (i, 0))
    spec1 = pl.BlockSpec((blk, D), lambda i: (i + NP // blk, 0))
    e1, e2, ssum = pl.pallas_call(
        _tc_body,
        grid=grid,
        in_specs=[spec0, spec0, spec1, spec0, spec1],
        out_specs=[spec0, spec0, spec0],
        out_shape=(
            jax.ShapeDtypeStruct((NP, D), jnp.float32),
            jax.ShapeDtypeStruct((NP, D), jnp.float32),
            jax.ShapeDtypeStruct((NP, D), jnp.float32),
        ),
    )(emb_p, p_out, p_out, q_out, q_out)
    return e1, e2, ssum


def kernel(edge_index, edge_values, item_emb):
    row = edge_index[0].astype(jnp.int32)
    col = edge_index[1].astype(jnp.int32)
    row4a = row.reshape(NC * NS, NSUP1, SUP, C)
    col4a = col.reshape(NC * NS, NSUP1, SUP, C)
    row4b = row.reshape(NS, NSUP2, SUP, C)
    col4b = col.reshape(NS, NSUP2, SUP, C)

    emb_p = jnp.concatenate(
        [item_emb, jnp.zeros((NP - N, D), jnp.float32)], axis=0)
    zeros = jnp.zeros((NP, D), jnp.float32)

    e1, e2, ssum = _run(emb_p, row4a, col4a, row4b, col4b,
                        edge_values, zeros)
    return (ssum[:N], item_emb, e1[:N], e2[:N])


# speedup vs baseline: 3.7831x; 1.0321x over previous
import jax
import jax.numpy as jnp
from jax import lax
from jax.experimental import pallas as pl
from jax.experimental.pallas import tpu as pltpu
from jax.experimental.pallas import tpu_sc as plsc

N = 10001
D = 128
E = 320000
NP = 10240
NC = 2
NS = 16
C = 80
SUP = 25
SUPE = SUP * C
EW1 = E // (NC * NS)
NSUP1 = EW1 // SUPE
EW2 = E // NS
NSUP2 = EW2 // SUPE
NPS = NP // NS
L = 16


def _sc_body(emb_p, row4a, col4a, row4b, col4b, vals, zeros,
             p_out, q_out, acc, rowb, colb, valb, gbuf, sem):
    c = lax.axis_index("c")
    s = lax.axis_index("s")
    w = c * NS + s
    rbase = s * NPS
    half = c * NP

    pltpu.sync_copy(zeros.at[pl.ds(rbase, NPS)], acc.at[pl.ds(rbase, NPS)])
    plsc.subcore_barrier()

    offv = jnp.broadcast_to(half, (L,)).astype(jnp.int32)

    def spmm_phase(row4, col4, widx, vbase, nsup, table, add_off):
        def sup(m, _):
            pltpu.sync_copy(row4.at[widx, m], rowb)
            pltpu.sync_copy(col4.at[widx, m], colb)
            pltpu.sync_copy(vals.at[pl.ds(vbase + m * SUPE, SUPE)],
                            valb.at[pl.ds(0, SUPE)])

            if add_off:
                def adj_row(k, _):
                    def adj_grp(j, _):
                        colb[k, pl.ds(j * L, L)] = (
                            colb[k, pl.ds(j * L, L)] + offv)
                        return 0
                    return lax.fori_loop(0, C // L, adj_grp, 0)
                lax.fori_loop(0, SUP, adj_row, 0)

            def chunk(k, _):
                pltpu.async_copy(table.at[colb.at[k]], gbuf, sem).wait()

                def edge(i, _):
                    vv = jnp.broadcast_to(valb[pl.ds(k * C + i, L)][0], (L,))
                    for j in range(D // L):
                        gbuf[i, pl.ds(j * L, L)] = (
                            gbuf[i, pl.ds(j * L, L)] * vv)
                    return 0

                lax.fori_loop(0, C, edge, 0, unroll=4)

                pltpu.sync_copy(gbuf, acc.at[rowb.at[k]], add=True)
                return 0

            lax.fori_loop(0, SUP, chunk, 0)
            return 0

        lax.fori_loop(0, nsup, sup, 0)
        plsc.subcore_barrier()

    spmm_phase(row4a, col4a, w, w * EW1, NSUP1, emb_p, False)

    pltpu.sync_copy(acc.at[pl.ds(rbase, NPS)],
                    p_out.at[pl.ds(half + rbase, NPS)])
    pltpu.sync_copy(zeros.at[pl.ds(rbase, NPS)], acc.at[pl.ds(rbase, NPS)])
    plsc.subcore_barrier()

    spmm_phase(row4b, col4b, s, s * EW2, NSUP2, p_out, True)

    pltpu.sync_copy(acc.at[pl.ds(rbase, NPS)],
                    q_out.at[pl.ds(half + rbase, NPS)])


def _tc_body(e0_ref, p0_ref, p1_ref, q0_ref, q1_ref,
             e1_ref, e2_ref, sum_ref):
    e1 = p0_ref[...] + p1_ref[...]
    e2 = q0_ref[...] + q1_ref[...]
    e1_ref[...] = e1
    e2_ref[...] = e2
    sum_ref[...] = e0_ref[...] + e1 + e2


@jax.jit
def _run(emb_p, row4a, col4a, row4b, col4b, vals, zeros):
    mesh = plsc.VectorSubcoreMesh(core_axis_name="c", subcore_axis_name="s")
    sc = pl.kernel(
        _sc_body,
        out_type=(
            jax.ShapeDtypeStruct((NC * NP, D), jnp.float32),
            jax.ShapeDtypeStruct((NC * NP, D), jnp.float32),
        ),
        mesh=mesh,
        scratch_types=[
            pltpu.VMEM_SHARED((NP, D), jnp.float32),
            pltpu.VMEM((SUP, C), jnp.int32),
            pltpu.VMEM((SUP, C), jnp.int32),
            pltpu.VMEM((SUPE + L,), jnp.float32),
            pltpu.VMEM((C, D), jnp.float32),
            pltpu.SemaphoreType.DMA,
        ],
    )
    p_out, q_out = sc(emb_p, row4a, col4a, row4b, col4b, vals, zeros)

    blk = 512
    grid = (NP // blk,)
    spec0 = pl.BlockSpec((blk, D), lambda i: (i, 0))
    spec1 = pl.BlockSpec((blk, D), lambda i: (i + NP // blk, 0))
    e1, e2, ssum = pl.pallas_call(
        _tc_body,
        grid=grid,
        in_specs=[spec0, spec0, spec1, spec0, spec1],
        out_specs=[spec0, spec0, spec0],
        out_shape=(
            jax.ShapeDtypeStruct((NP, D), jnp.float32),
            jax.ShapeDtypeStruct((NP, D), jnp.float32),
            jax.ShapeDtypeStruct((NP, D), jnp.float32),
        ),
    )(emb_p, p_out, p_out, q_out, q_out)
    return e1, e2, ssum


def kernel(edge_index, edge_values, item_emb):
    row = edge_index[0].astype(jnp.int32)
    col = edge_index[1].astype(jnp.int32)
    row4a = row.reshape(NC * NS, NSUP1, SUP, C)
    col4a = col.reshape(NC * NS, NSUP1, SUP, C)
    row4b = row.reshape(NS, NSUP2, SUP, C)
    col4b = col.reshape(NS, NSUP2, SUP, C)

    emb_p = jnp.concatenate(
        [item_emb, jnp.zeros((NP - N, D), jnp.float32)], axis=0)
    zeros = jnp.zeros((NP, D), jnp.float32)

    e1, e2, ssum = _run(emb_p, row4a, col4a, row4b, col4b,
                        edge_values, zeros)
    return (ssum[:N], item_emb, e1[:N], e2[:N])


# speedup vs baseline: 6.2225x; 1.6448x over previous
import jax
import jax.numpy as jnp
from jax import lax
from jax.experimental import pallas as pl
from jax.experimental.pallas import tpu as pltpu
from jax.experimental.pallas import tpu_sc as plsc

N = 10001
D = 128
E = 320000
NP = 10240
NC = 2
NS = 16
C = 100
SUP = 20
SUPE = SUP * C
EW1 = E // (NC * NS)
NSUP1 = EW1 // SUPE
EW2 = E // NS
NSUP2 = EW2 // SUPE
NPS = NP // NS
L = 16


def _sc_body(emb_p, row4a, col4a, row4b, col4b_lo, col4b_hi, vals, zeros,
             p_out, q_out, acc, rowb, colb, valb, g0, g1, sem0, sem1):
    c = lax.axis_index("c")
    s = lax.axis_index("s")
    w = c * NS + s
    rbase = s * NPS
    half = c * NP

    pltpu.sync_copy(zeros.at[pl.ds(rbase, NPS)], acc.at[pl.ds(rbase, NPS)])
    plsc.subcore_barrier()

    def spmm_phase(row4, col4_0, col4_1, widx, vbase, nsup, table):
        def sup(m, _):
            pltpu.sync_copy(row4.at[widx, m], rowb)
            if col4_1 is None:
                pltpu.sync_copy(col4_0.at[widx, m], colb)
            else:
                @pl.when(c == 0)
                def _():
                    pltpu.sync_copy(col4_0.at[widx, m], colb)

                @pl.when(c == 1)
                def _():
                    pltpu.sync_copy(col4_1.at[widx, m], colb)

            pltpu.sync_copy(vals.at[pl.ds(vbase + m * SUPE, SUPE)],
                            valb.at[pl.ds(0, SUPE)])

            def g_start(k, buf, sem):
                pltpu.async_copy(table.at[colb.at[k]], buf, sem)

            def g_wait(k, buf, sem):
                pltpu.make_async_copy(table.at[colb.at[k]], buf, sem).wait()

            def scale_scatter(k, buf):
                def edge(i, _):
                    vv = jnp.broadcast_to(valb[pl.ds(k * C + i, L)][0], (L,))
                    for j in range(D // L):
                        buf[i, pl.ds(j * L, L)] = (
                            buf[i, pl.ds(j * L, L)] * vv)
                    return 0

                lax.fori_loop(0, C, edge, 0, unroll=4)

                pltpu.sync_copy(buf, acc.at[rowb.at[k]], add=True)

            g_start(0, g0, sem0)

            def pair(t, _):
                k0 = 2 * t
                k1 = k0 + 1
                g_start(k1, g1, sem1)
                g_wait(k0, g0, sem0)
                scale_scatter(k0, g0)

                @pl.when(t < SUP // 2 - 1)
                def _():
                    g_start(k0 + 2, g0, sem0)

                g_wait(k1, g1, sem1)
                scale_scatter(k1, g1)
                return 0

            lax.fori_loop(0, SUP // 2, pair, 0)
            return 0

        lax.fori_loop(0, nsup, sup, 0)
        plsc.subcore_barrier()

    spmm_phase(row4a, col4a, None, w, w * EW1, NSUP1, emb_p)

    pltpu.sync_copy(acc.at[pl.ds(rbase, NPS)],
                    p_out.at[pl.ds(half + rbase, NPS)])
    pltpu.sync_copy(zeros.at[pl.ds(rbase, NPS)], acc.at[pl.ds(rbase, NPS)])
    plsc.subcore_barrier()

    spmm_phase(row4b, col4b_lo, col4b_hi, s, s * EW2, NSUP2, p_out)

    pltpu.sync_copy(acc.at[pl.ds(rbase, NPS)],
                    q_out.at[pl.ds(half + rbase, NPS)])


def _tc_body(e0_ref, p0_ref, p1_ref, q0_ref, q1_ref,
             e1_ref, e2_ref, sum_ref):
    e1 = p0_ref[...] + p1_ref[...]
    e2 = q0_ref[...] + q1_ref[...]
    e1_ref[...] = e1
    e2_ref[...] = e2
    sum_ref[...] = e0_ref[...] + e1 + e2


@jax.jit
def _run(emb_p, row4a, col4a, row4b, col4b_lo, col4b_hi, vals, zeros):
    mesh = plsc.VectorSubcoreMesh(core_axis_name="c", subcore_axis_name="s")
    sc = pl.kernel(
        _sc_body,
        out_type=(
            jax.ShapeDtypeStruct((NC * NP, D), jnp.float32),
            jax.ShapeDtypeStruct((NC * NP, D), jnp.float32),
        ),
        mesh=mesh,
        scratch_types=[
            pltpu.VMEM_SHARED((NP, D), jnp.float32),
            pltpu.VMEM((SUP, C), jnp.int32),
            pltpu.VMEM((SUP, C), jnp.int32),
            pltpu.VMEM((SUPE + L,), jnp.float32),
            pltpu.VMEM((C, D), jnp.float32),
            pltpu.VMEM((C, D), jnp.float32),
            pltpu.SemaphoreType.DMA,
            pltpu.SemaphoreType.DMA,
        ],
    )
    p_out, q_out = sc(emb_p, row4a, col4a, row4b, col4b_lo, col4b_hi,
                      vals, zeros)

    blk = 512
    grid = (NP // blk,)
    spec0 = pl.BlockSpec((blk, D), lambda i: (i, 0))
    spec1 = pl.BlockSpec((blk, D), lambda i: (i + NP // blk, 0))
    e1, e2, ssum = pl.pallas_call(
        _tc_body,
        grid=grid,
        in_specs=[spec0, spec0, spec1, spec0, spec1],
        out_specs=[spec0, spec0, spec0],
        out_shape=(
            jax.ShapeDtypeStruct((NP, D), jnp.float32),
            jax.ShapeDtypeStruct((NP, D), jnp.float32),
            jax.ShapeDtypeStruct((NP, D), jnp.float32),
        ),
    )(emb_p, p_out, p_out, q_out, q_out)
    return e1, e2, ssum


def kernel(edge_index, edge_values, item_emb):
    row = edge_index[0].astype(jnp.int32)
    col = edge_index[1].astype(jnp.int32)
    row4a = row.reshape(NC * NS, NSUP1, SUP, C)
    col4a = col.reshape(NC * NS, NSUP1, SUP, C)
    row4b = row.reshape(NS, NSUP2, SUP, C)
    col4b_lo = col.reshape(NS, NSUP2, SUP, C)
    col4b_hi = (col + NP).reshape(NS, NSUP2, SUP, C)

    emb_p = jnp.concatenate(
        [item_emb, jnp.zeros((NP - N, D), jnp.float32)], axis=0)
    zeros = jnp.zeros((NP, D), jnp.float32)

    e1, e2, ssum = _run(emb_p, row4a, col4a, row4b, col4b_lo, col4b_hi,
                        edge_values, zeros)
    return (ssum[:N], item_emb, e1[:N], e2[:N])
